# baseline (speedup 1.0000x reference)
import functools

import jax
import jax.numpy as jnp
from jax import lax
from jax.experimental import pallas as pl
from jax.experimental.pallas import tpu as pltpu
from jax.experimental.pallas import tpu_sc as plsc

N = 10000
E = 320000
D = 128
OUT = 128
L = 2
EPS = 0.3

NC = 2
NS = 16
NW = NC * NS
LANES = 16
EPW = E // NW
EPT = E // NS

K = 80
NCHUNK = EPW // K
NBUF = 4

RT_BIG = 640
RT_SMALL = N - 15 * RT_BIG
ZR = 40
N_PAD = 10240
ND_T = N_PAD // NS

_mesh = plsc.VectorSubcoreMesh(core_axis_name="c", subcore_axis_name="s")
_sc_params = pltpu.CompilerParams(needs_layout_passes=False)


@functools.partial(
    pl.kernel,
    out_type=jax.ShapeDtypeStruct((N_PAD,), jnp.float32),
    mesh=_mesh,
    scratch_types=[
        pltpu.VMEM((EPT,), jnp.int32),
        pltpu.VMEM((K,), jnp.float32),
        pltpu.VMEM((K,), jnp.int32),
        pltpu.VMEM((K,), jnp.int32),
        pltpu.VMEM((ND_T,), jnp.float32),
        pltpu.VMEM_SHARED((N_PAD,), jnp.float32),
        pltpu.SemaphoreType.DMA((2,)),
    ],
    compiler_params=_sc_params,
)
def _deg_kernel(row_hbm, deg_hbm, row1d, ones_v, didx, didx2, work_v, deg_sh,
                dsem):
    cid = lax.axis_index("c")
    sid = lax.axis_index("s")

    @pl.when(cid == 0)
    def _():
        pltpu.sync_copy(row_hbm.at[pl.ds(sid * EPT, EPT)], row1d)

        zv = jnp.zeros((LANES,), jnp.float32)
        ov = jnp.ones((LANES,), jnp.float32)

        def fill_zero(t, _):
            work_v[pl.ds(t * LANES, LANES)] = zv
            return 0

        lax.fori_loop(0, ND_T // LANES, fill_zero, 0)
        pltpu.sync_copy(work_v, deg_sh.at[pl.ds(sid * ND_T, ND_T)])

        for t in range(K // LANES):
            ones_v[pl.ds(t * LANES, LANES)] = ov

        plsc.subcore_barrier()

        didxs = [didx, didx2]

        def scat(j2, _):
            for b in range(2):
                j = 2 * j2 + b

                @pl.when(j >= 2)
                def _():
                    pltpu.make_async_copy(ones_v, deg_sh.at[didxs[b]],
                                          dsem.at[b]).wait()

                for t in range(K // LANES):
                    didxs[b][pl.ds(t * LANES, LANES)] = (
                        row1d[pl.ds(j * K + t * LANES, LANES)])
                pltpu.async_copy(ones_v, deg_sh.at[didxs[b]], dsem.at[b],
                                 add=True)
            return 0

        lax.fori_loop(0, EPT // K // 2, scat, 0)
        for b in range(2):
            pltpu.make_async_copy(ones_v, deg_sh.at[didxs[b]],
                                  dsem.at[b]).wait()

        plsc.subcore_barrier()

        pltpu.sync_copy(deg_sh.at[pl.ds(sid * ND_T, ND_T)],
                        deg_hbm.at[pl.ds(sid * ND_T, ND_T)])


@functools.partial(
    pl.kernel,
    out_type=jax.ShapeDtypeStruct((E,), jnp.float32),
    mesh=_mesh,
    scratch_types=[
        pltpu.VMEM((2, N), jnp.float32),
        pltpu.VMEM((N,), jnp.float32),
        pltpu.VMEM((EPW,), jnp.int32),
        pltpu.VMEM((EPW,), jnp.int32),
        pltpu.VMEM((EPW,), jnp.float32),
    ],
    compiler_params=_sc_params,
)
def _coeff_kernel(ab_hbm, nd_hbm, row_hbm, col_hbm, co_hbm,
                  ab_v, nd_v, row_v, col_v, co_v):
    cid = lax.axis_index("c")
    sid = lax.axis_index("s")
    wid = cid * NS + sid
    ebase = wid * EPW

    pltpu.sync_copy(ab_hbm, ab_v)
    pltpu.sync_copy(nd_hbm.at[pl.ds(0, N)], nd_v)
    pltpu.sync_copy(row_hbm.at[pl.ds(ebase, EPW)], row_v)
    pltpu.sync_copy(col_hbm.at[pl.ds(ebase, EPW)], col_v)

    @plsc.parallel_loop(0, EPW // LANES, unroll=4)
    def _(t):
        ridx = row_v[pl.ds(t * LANES, LANES)]
        cidx = col_v[pl.ds(t * LANES, LANES)]
        zero16 = ridx - ridx
        av = plsc.load_gather(ab_v, [zero16, ridx])
        bv = plsc.load_gather(ab_v, [zero16 + 1, cidx])
        nr = plsc.load_gather(nd_v, [ridx])
        ncv = plsc.load_gather(nd_v, [cidx])
        s = av + bv
        s = jnp.minimum(jnp.maximum(s, -10.0), 10.0)
        e2 = jnp.exp(2.0 * s)
        co_v[pl.ds(t * LANES, LANES)] = (e2 - 1.0) / (e2 + 1.0) * nr * ncv
    pltpu.sync_copy(co_v, co_hbm.at[pl.ds(ebase, EPW)])


@functools.partial(
    pl.kernel,
    out_type=jax.ShapeDtypeStruct((NC, N, D), jnp.float32),
    mesh=_mesh,
    scratch_types=[
        [pltpu.VMEM((K,), jnp.int32) for _ in range(NBUF)],
        [pltpu.VMEM((K,), jnp.int32) for _ in range(NBUF)],
        [pltpu.VMEM((K,), jnp.float32) for _ in range(NBUF)],
        pltpu.VMEM((NBUF, K, D), jnp.float32),
        pltpu.VMEM((ZR, D), jnp.float32),
        pltpu.VMEM_SHARED((N, D), jnp.float32),
        pltpu.SemaphoreType.DMA((NBUF,)),
        pltpu.SemaphoreType.DMA((NBUF,)),
        pltpu.SemaphoreType.DMA((NBUF,)),
    ],
    compiler_params=_sc_params,
)
def _mp_kernel(h_hbm, co_hbm, row_hbm, col_hbm, out_hbm,
               rbufs, cbufs, fbufs, hbuf, zbuf, acc, isem, gsem, ssem):
    cid = lax.axis_index("c")
    sid = lax.axis_index("s")
    wid = cid * NS + sid
    ebase = wid * EPW

    zv = jnp.zeros((LANES,), jnp.float32)

    def fill_zero(r, _):
        for q in range(D // LANES):
            zbuf[r, pl.ds(q * LANES, LANES)] = zv
        return 0

    lax.fori_loop(0, ZR, fill_zero, 0)

    @pl.when(sid < 15)
    def _():
        for kz in range(RT_BIG // ZR):
            pltpu.sync_copy(zbuf, acc.at[pl.ds(sid * RT_BIG + kz * ZR, ZR)])

    @pl.when(sid == 15)
    def _():
        for kz in range(RT_SMALL // ZR):
            pltpu.sync_copy(zbuf, acc.at[pl.ds(15 * RT_BIG + kz * ZR, ZR)])

    plsc.subcore_barrier()

    def start_idx(c, p):
        off = ebase + c * K
        pltpu.async_copy(row_hbm.at[pl.ds(off, K)], rbufs[p], isem.at[p])
        pltpu.async_copy(col_hbm.at[pl.ds(off, K)], cbufs[p], isem.at[p])
        pltpu.async_copy(co_hbm.at[pl.ds(off, K)], fbufs[p], isem.at[p])

    def wait_idx(p):
        pltpu.make_async_copy(row_hbm.at[pl.ds(0, K)], rbufs[p],
                              isem.at[p]).wait()
        pltpu.make_async_copy(col_hbm.at[pl.ds(0, K)], cbufs[p],
                              isem.at[p]).wait()
        pltpu.make_async_copy(co_hbm.at[pl.ds(0, K)], fbufs[p],
                              isem.at[p]).wait()

    def start_gather(p):
        pltpu.async_copy(h_hbm.at[rbufs[p]], hbuf.at[p], gsem.at[p])

    def wait_gather(p):
        pltpu.make_async_copy(h_hbm.at[rbufs[p]], hbuf.at[p],
                              gsem.at[p]).wait()

    def start_scatter(p):
        pltpu.async_copy(hbuf.at[p], acc.at[cbufs[p]], ssem.at[p], add=True)

    def wait_scatter(p):
        pltpu.make_async_copy(hbuf.at[p], acc.at[cbufs[p]], ssem.at[p]).wait()

    def compute_chunk(p):
        hb = hbuf.at[p]
        fb = fbufs[p]

        @plsc.parallel_loop(0, K // LANES)
        def _(t):
            coeff = fb[pl.ds(t * LANES, LANES)]
            for i in range(LANES):
                e = t * LANES + i
                sc = coeff[i]
                for q in range(D // LANES):
                    sl = pl.ds(q * LANES, LANES)
                    hb[e, sl] = hb[e, sl] * sc

    start_idx(0, 0)
    start_idx(1, 1)
    wait_idx(0)
    start_gather(0)

    def outer(i4, _):
        for p in range(NBUF):
            c = i4 * NBUF + p

            @pl.when(c + 2 <= NCHUNK - 1)
            def _():
                @pl.when(c >= 2)
                def _():
                    wait_scatter((p + 2) % NBUF)

                start_idx(c + 2, (p + 2) % NBUF)

            wait_idx((p + 1) % NBUF)
            start_gather((p + 1) % NBUF)

            wait_gather(p)
            compute_chunk(p)
            start_scatter(p)
        return 0

    lax.fori_loop(0, (NCHUNK - 1) // NBUF, outer, 0)

    wait_gather(0)
    compute_chunk(0)
    start_scatter(0)
    for p in range(NBUF):
        wait_scatter((1 + p) % NBUF)

    plsc.subcore_barrier()

    @pl.when(sid < 15)
    def _():
        pltpu.sync_copy(acc.at[pl.ds(sid * RT_BIG, RT_BIG)],
                        out_hbm.at[cid].at[pl.ds(sid * RT_BIG, RT_BIG)])

    @pl.when(sid == 15)
    def _():
        pltpu.sync_copy(acc.at[pl.ds(15 * RT_BIG, RT_SMALL)],
                        out_hbm.at[cid].at[pl.ds(15 * RT_BIG, RT_SMALL)])


def _prep0_body(x_ref, w_ref, bias_ref, deg_ref, ab_ref, nd_ref):
    ab = lax.dot_general(w_ref[...], x_ref[...], (((0,), (1,)), ((), ())),
                         preferred_element_type=jnp.float32)
    ab_ref[...] = ab + bias_ref[...][:, :1]
    nd_ref[...] = lax.rsqrt(jnp.maximum(deg_ref[...], 1.0))


def _prep1_body(x_ref, part_ref, w_ref, bias_ref, h_ref, ab_ref):
    h = EPS * x_ref[...] + part_ref[0] + part_ref[1]
    h_ref[...] = h
    ab = lax.dot_general(w_ref[...], h, (((0,), (1,)), ((), ())),
                         preferred_element_type=jnp.float32)
    ab_ref[...] = ab + bias_ref[...][:, :1]


def _final_body(x_ref, part_ref, w_ref, bias_ref, out_ref):
    h = EPS * x_ref[...] + part_ref[0] + part_ref[1]
    out_ref[...] = jnp.dot(h, w_ref[...],
                           preferred_element_type=jnp.float32) + bias_ref[...]


def _prep0(x, wcat, bias2, deg2d):
    return pl.pallas_call(
        _prep0_body,
        out_shape=[
            jax.ShapeDtypeStruct((2, N), jnp.float32),
            jax.ShapeDtypeStruct((N_PAD // 128, 128), jnp.float32),
        ],
    )(x, wcat, bias2, deg2d)


def _prep1(x, part, wcat, bias2):
    return pl.pallas_call(
        _prep1_body,
        out_shape=[
            jax.ShapeDtypeStruct((N, D), jnp.float32),
            jax.ShapeDtypeStruct((2, N), jnp.float32),
        ],
    )(x, part, wcat, bias2)


def _final(x, part, w_head, bias):
    return pl.pallas_call(
        _final_body,
        out_shape=jax.ShapeDtypeStruct((N, OUT), jnp.float32),
    )(x, part, w_head, bias)


def kernel(x, edge_index, W_gate, b_gate, W_head, b_head):
    row = edge_index[0]
    col = edge_index[1]

    deg = _deg_kernel(row)
    deg2d = deg.reshape(N_PAD // 128, 128)

    wg1 = W_gate[:, :D, 0]
    wg2 = W_gate[:, D:, 0]

    part = None
    nd = None
    h = x
    for i in range(L):
        wcat = jnp.stack([wg1[i], wg2[i]], axis=1)
        bias2 = jnp.broadcast_to(
            jnp.concatenate([b_gate[i], jnp.zeros((1,), jnp.float32)])[:, None],
            (2, 128))
        if i == 0:
            ab, nd2d = _prep0(x, wcat, bias2, deg2d)
            nd = nd2d.reshape(N_PAD)
        else:
            h, ab = _prep1(x, part, wcat, bias2)
        co = _coeff_kernel(ab, nd, row, col)
        part = _mp_kernel(h, co, row, col)

    return _final(x, part, W_head, jnp.reshape(b_head, (1, OUT)))


# speedup vs baseline: 1.1772x; 1.1772x over previous
import functools

import jax
import jax.numpy as jnp
from jax import lax
from jax.experimental import pallas as pl
from jax.experimental.pallas import tpu as pltpu
from jax.experimental.pallas import tpu_sc as plsc

N = 10000
E = 320000
D = 128
OUT = 128
L = 2
EPS = 0.3

NC = 2
NS = 16
NW = NC * NS
LANES = 16
EPW = E // NW
EPT = E // NS

K = 80
NCHUNK = EPW // K
NBUF = 4

RT_BIG = 640
RT_SMALL = N - 15 * RT_BIG
ZR = 40
N_PAD = 10240
ND_T = N_PAD // NS

_mesh = plsc.VectorSubcoreMesh(core_axis_name="c", subcore_axis_name="s")
_sc_params = pltpu.CompilerParams(needs_layout_passes=False)


def _coeff_phase(ab_hbm, row_hbm, col_hbm, co_hbm, ab_v, nd_v, row_v, col_v,
                 co_v, ebase):
    pltpu.sync_copy(ab_hbm, ab_v)
    pltpu.sync_copy(row_hbm.at[pl.ds(ebase, EPW)], row_v)
    pltpu.sync_copy(col_hbm.at[pl.ds(ebase, EPW)], col_v)

    @plsc.parallel_loop(0, EPW // LANES, unroll=4)
    def _(t):
        ridx = row_v[pl.ds(t * LANES, LANES)]
        cidx = col_v[pl.ds(t * LANES, LANES)]
        zero16 = ridx - ridx
        av = plsc.load_gather(ab_v, [zero16, ridx])
        bv = plsc.load_gather(ab_v, [zero16 + 1, cidx])
        nr = plsc.load_gather(nd_v, [ridx])
        ncv = plsc.load_gather(nd_v, [cidx])
        s = av + bv
        s = jnp.minimum(jnp.maximum(s, -10.0), 10.0)
        e2 = jnp.exp(2.0 * s)
        co_v[pl.ds(t * LANES, LANES)] = (e2 - 1.0) / (e2 + 1.0) * nr * ncv

    pltpu.sync_copy(co_v, co_hbm.at[pl.ds(ebase, EPW)])


@functools.partial(
    pl.kernel,
    out_type=[
        jax.ShapeDtypeStruct((E,), jnp.float32),
        jax.ShapeDtypeStruct((N_PAD,), jnp.float32),
    ],
    mesh=_mesh,
    scratch_types=[
        pltpu.VMEM((EPT,), jnp.int32),
        pltpu.VMEM((K,), jnp.float32),
        pltpu.VMEM((K,), jnp.int32),
        pltpu.VMEM((K,), jnp.int32),
        pltpu.VMEM((ND_T,), jnp.float32),
        pltpu.VMEM((2, N), jnp.float32),
        pltpu.VMEM((N_PAD,), jnp.float32),
        pltpu.VMEM((EPW,), jnp.int32),
        pltpu.VMEM((EPW,), jnp.int32),
        pltpu.VMEM((EPW,), jnp.float32),
        pltpu.VMEM_SHARED((N_PAD,), jnp.float32),
        pltpu.SemaphoreType.DMA((2,)),
    ],
    compiler_params=_sc_params,
)
def _coeff0_kernel(ab_hbm, row_hbm, col_hbm, co_hbm, nd_hbm,
                   row1d, ones_v, didx, didx2, work_v,
                   ab_v, nd_v, row_v, col_v, co_v, deg_sh, dsem):
    cid = lax.axis_index("c")
    sid = lax.axis_index("s")
    wid = cid * NS + sid

    pltpu.sync_copy(row_hbm.at[pl.ds(sid * EPT, EPT)], row1d)

    zv = jnp.zeros((LANES,), jnp.float32)
    ov = jnp.ones((LANES,), jnp.float32)

    def fill_zero(t, _):
        work_v[pl.ds(t * LANES, LANES)] = zv
        return 0

    lax.fori_loop(0, ND_T // LANES, fill_zero, 0)
    pltpu.sync_copy(work_v, deg_sh.at[pl.ds(sid * ND_T, ND_T)])

    for t in range(K // LANES):
        ones_v[pl.ds(t * LANES, LANES)] = ov

    plsc.subcore_barrier()

    didxs = [didx, didx2]

    def scat(j2, _):
        for b in range(2):
            j = 2 * j2 + b

            @pl.when(j >= 2)
            def _():
                pltpu.make_async_copy(ones_v, deg_sh.at[didxs[b]],
                                      dsem.at[b]).wait()

            for t in range(K // LANES):
                didxs[b][pl.ds(t * LANES, LANES)] = (
                    row1d[pl.ds(j * K + t * LANES, LANES)])
            pltpu.async_copy(ones_v, deg_sh.at[didxs[b]], dsem.at[b],
                             add=True)
        return 0

    lax.fori_loop(0, EPT // K // 2, scat, 0)
    for b in range(2):
        pltpu.make_async_copy(ones_v, deg_sh.at[didxs[b]], dsem.at[b]).wait()

    plsc.subcore_barrier()

    pltpu.sync_copy(deg_sh.at[pl.ds(sid * ND_T, ND_T)], work_v)

    def newton(t, _):
        x = jnp.maximum(work_v[pl.ds(t * LANES, LANES)], 1.0)
        tt = 1.0 / x
        z = jnp.full((LANES,), 0.25, jnp.float32) + 0.0 * tt
        for _ in range(6):
            z = 0.5 * (z + tt / z)
        work_v[pl.ds(t * LANES, LANES)] = z
        return 0

    lax.fori_loop(0, ND_T // LANES, newton, 0)
    pltpu.sync_copy(work_v, deg_sh.at[pl.ds(sid * ND_T, ND_T)])

    @pl.when(cid == 0)
    def _():
        pltpu.sync_copy(work_v, nd_hbm.at[pl.ds(sid * ND_T, ND_T)])

    plsc.subcore_barrier()
    pltpu.sync_copy(deg_sh, nd_v)

    _coeff_phase(ab_hbm, row_hbm, col_hbm, co_hbm, ab_v, nd_v, row_v, col_v,
                 co_v, wid * EPW)


@functools.partial(
    pl.kernel,
    out_type=jax.ShapeDtypeStruct((E,), jnp.float32),
    mesh=_mesh,
    scratch_types=[
        pltpu.VMEM((2, N), jnp.float32),
        pltpu.VMEM((N_PAD,), jnp.float32),
        pltpu.VMEM((EPW,), jnp.int32),
        pltpu.VMEM((EPW,), jnp.int32),
        pltpu.VMEM((EPW,), jnp.float32),
    ],
    compiler_params=_sc_params,
)
def _coeff_kernel(ab_hbm, nd_hbm, row_hbm, col_hbm, co_hbm,
                  ab_v, nd_v, row_v, col_v, co_v):
    cid = lax.axis_index("c")
    sid = lax.axis_index("s")
    wid = cid * NS + sid

    pltpu.sync_copy(nd_hbm, nd_v)
    _coeff_phase(ab_hbm, row_hbm, col_hbm, co_hbm, ab_v, nd_v, row_v, col_v,
                 co_v, wid * EPW)


@functools.partial(
    pl.kernel,
    out_type=jax.ShapeDtypeStruct((NC, N, D), jnp.float32),
    mesh=_mesh,
    scratch_types=[
        [pltpu.VMEM((K,), jnp.int32) for _ in range(NBUF)],
        [pltpu.VMEM((K,), jnp.int32) for _ in range(NBUF)],
        [pltpu.VMEM((K,), jnp.float32) for _ in range(NBUF)],
        pltpu.VMEM((NBUF, K, D), jnp.float32),
        pltpu.VMEM((ZR, D), jnp.float32),
        pltpu.VMEM_SHARED((N, D), jnp.float32),
        pltpu.SemaphoreType.DMA((NBUF,)),
        pltpu.SemaphoreType.DMA((NBUF,)),
        pltpu.SemaphoreType.DMA((NBUF,)),
    ],
    compiler_params=_sc_params,
)
def _mp_kernel(h_hbm, co_hbm, row_hbm, col_hbm, out_hbm,
               rbufs, cbufs, fbufs, hbuf, zbuf, acc, isem, gsem, ssem):
    cid = lax.axis_index("c")
    sid = lax.axis_index("s")
    wid = cid * NS + sid
    ebase = wid * EPW

    zv = jnp.zeros((LANES,), jnp.float32)

    def fill_zero(r, _):
        for q in range(D // LANES):
            zbuf[r, pl.ds(q * LANES, LANES)] = zv
        return 0

    lax.fori_loop(0, ZR, fill_zero, 0)

    @pl.when(sid < 15)
    def _():
        for kz in range(RT_BIG // ZR):
            pltpu.sync_copy(zbuf, acc.at[pl.ds(sid * RT_BIG + kz * ZR, ZR)])

    @pl.when(sid == 15)
    def _():
        for kz in range(RT_SMALL // ZR):
            pltpu.sync_copy(zbuf, acc.at[pl.ds(15 * RT_BIG + kz * ZR, ZR)])

    plsc.subcore_barrier()

    def start_idx(c, p):
        off = ebase + c * K
        pltpu.async_copy(row_hbm.at[pl.ds(off, K)], rbufs[p], isem.at[p])
        pltpu.async_copy(col_hbm.at[pl.ds(off, K)], cbufs[p], isem.at[p])
        pltpu.async_copy(co_hbm.at[pl.ds(off, K)], fbufs[p], isem.at[p])

    def wait_idx(p):
        pltpu.make_async_copy(row_hbm.at[pl.ds(0, K)], rbufs[p],
                              isem.at[p]).wait()
        pltpu.make_async_copy(col_hbm.at[pl.ds(0, K)], cbufs[p],
                              isem.at[p]).wait()
        pltpu.make_async_copy(co_hbm.at[pl.ds(0, K)], fbufs[p],
                              isem.at[p]).wait()

    def start_gather(p):
        pltpu.async_copy(h_hbm.at[rbufs[p]], hbuf.at[p], gsem.at[p])

    def wait_gather(p):
        pltpu.make_async_copy(h_hbm.at[rbufs[p]], hbuf.at[p],
                              gsem.at[p]).wait()

    def start_scatter(p):
        pltpu.async_copy(hbuf.at[p], acc.at[cbufs[p]], ssem.at[p], add=True)

    def wait_scatter(p):
        pltpu.make_async_copy(hbuf.at[p], acc.at[cbufs[p]], ssem.at[p]).wait()

    def compute_chunk(p):
        hb = hbuf.at[p]
        fb = fbufs[p]

        def group(t, _):
            coeff = fb[pl.ds(t * LANES, LANES)]
            for i in range(LANES):
                e = t * LANES + i
                sc = coeff[i]
                for q in range(D // LANES):
                    sl = pl.ds(q * LANES, LANES)
                    hb[e, sl] = hb[e, sl] * sc
            return 0

        lax.fori_loop(0, K // LANES, group, 0)

    start_idx(0, 0)
    start_idx(1, 1)
    wait_idx(0)
    start_gather(0)

    def outer(i4, _):
        for p in range(NBUF):
            c = i4 * NBUF + p

            @pl.when(c + 2 <= NCHUNK - 1)
            def _():
                @pl.when(c >= 2)
                def _():
                    wait_scatter((p + 2) % NBUF)

                start_idx(c + 2, (p + 2) % NBUF)

            wait_idx((p + 1) % NBUF)
            start_gather((p + 1) % NBUF)

            wait_gather(p)
            compute_chunk(p)
            start_scatter(p)
        return 0

    lax.fori_loop(0, (NCHUNK - 1) // NBUF, outer, 0)

    wait_gather(0)
    compute_chunk(0)
    start_scatter(0)
    for p in range(NBUF):
        wait_scatter((1 + p) % NBUF)

    plsc.subcore_barrier()

    @pl.when(sid < 15)
    def _():
        pltpu.sync_copy(acc.at[pl.ds(sid * RT_BIG, RT_BIG)],
                        out_hbm.at[cid].at[pl.ds(sid * RT_BIG, RT_BIG)])

    @pl.when(sid == 15)
    def _():
        pltpu.sync_copy(acc.at[pl.ds(15 * RT_BIG, RT_SMALL)],
                        out_hbm.at[cid].at[pl.ds(15 * RT_BIG, RT_SMALL)])


def _prep0_body(x_ref, w_ref, bias_ref, ab_ref):
    ab = lax.dot_general(w_ref[...], x_ref[...], (((0,), (1,)), ((), ())),
                         preferred_element_type=jnp.float32)
    ab_ref[...] = ab + bias_ref[...][:, :1]


def _prep1_body(x_ref, part_ref, w_ref, bias_ref, h_ref, ab_ref):
    h = EPS * x_ref[...] + part_ref[0] + part_ref[1]
    h_ref[...] = h
    ab = lax.dot_general(w_ref[...], h, (((0,), (1,)), ((), ())),
                         preferred_element_type=jnp.float32)
    ab_ref[...] = ab + bias_ref[...][:, :1]


def _final_body(x_ref, part_ref, w_ref, bias_ref, out_ref):
    h = EPS * x_ref[...] + part_ref[0] + part_ref[1]
    out_ref[...] = jnp.dot(h, w_ref[...],
                           preferred_element_type=jnp.float32) + bias_ref[...]


def _prep0(x, wcat, bias2):
    return pl.pallas_call(
        _prep0_body,
        out_shape=jax.ShapeDtypeStruct((2, N), jnp.float32),
    )(x, wcat, bias2)


def _prep1(x, part, wcat, bias2):
    return pl.pallas_call(
        _prep1_body,
        out_shape=[
            jax.ShapeDtypeStruct((N, D), jnp.float32),
            jax.ShapeDtypeStruct((2, N), jnp.float32),
        ],
    )(x, part, wcat, bias2)


def _final(x, part, w_head, bias):
    return pl.pallas_call(
        _final_body,
        out_shape=jax.ShapeDtypeStruct((N, OUT), jnp.float32),
    )(x, part, w_head, bias)


def kernel(x, edge_index, W_gate, b_gate, W_head, b_head):
    row = edge_index[0]
    col = edge_index[1]

    wg1 = W_gate[:, :D, 0]
    wg2 = W_gate[:, D:, 0]

    part = None
    nd = None
    h = x
    for i in range(L):
        wcat = jnp.stack([wg1[i], wg2[i]], axis=1)
        bias2 = jnp.broadcast_to(
            jnp.concatenate([b_gate[i], jnp.zeros((1,), jnp.float32)])[:, None],
            (2, 128))
        if i == 0:
            ab = _prep0(x, wcat, bias2)
            co, nd = _coeff0_kernel(ab, row, col)
        else:
            h, ab = _prep1(x, part, wcat, bias2)
            co = _coeff_kernel(ab, nd, row, col)
        part = _mp_kernel(h, co, row, col)

    return _final(x, part, W_head, jnp.reshape(b_head, (1, OUT)))


# speedup vs baseline: 1.1855x; 1.0070x over previous
import functools

import jax
import jax.numpy as jnp
from jax import lax
from jax.experimental import pallas as pl
from jax.experimental.pallas import tpu as pltpu
from jax.experimental.pallas import tpu_sc as plsc

N = 10000
E = 320000
D = 128
OUT = 128
L = 2
EPS = 0.3

NC = 2
NS = 16
NW = NC * NS
LANES = 16
EPW = E // NW
EPT = E // NS

K = 80
NCHUNK = EPW // K
NBUF = 4

RT_BIG = 640
RT_SMALL = N - 15 * RT_BIG
ZR = 40
N_PAD = 10240
ND_T = N_PAD // NS

_mesh = plsc.VectorSubcoreMesh(core_axis_name="c", subcore_axis_name="s")
_sc_params = pltpu.CompilerParams(needs_layout_passes=False)


@functools.partial(
    pl.kernel,
    out_type=jax.ShapeDtypeStruct((N_PAD,), jnp.float32),
    mesh=_mesh,
    scratch_types=[
        pltpu.VMEM((EPT,), jnp.int32),
        pltpu.VMEM((K,), jnp.float32),
        pltpu.VMEM((K,), jnp.int32),
        pltpu.VMEM((K,), jnp.int32),
        pltpu.VMEM((ND_T,), jnp.float32),
        pltpu.VMEM_SHARED((N_PAD,), jnp.float32),
        pltpu.SemaphoreType.DMA((2,)),
    ],
    compiler_params=_sc_params,
)
def _deg_kernel(row_hbm, deg_hbm, row1d, ones_v, didx, didx2, work_v, deg_sh,
                dsem):
    cid = lax.axis_index("c")
    sid = lax.axis_index("s")

    @pl.when(cid == 0)
    def _():
        pltpu.sync_copy(row_hbm.at[pl.ds(sid * EPT, EPT)], row1d)

        zv = jnp.zeros((LANES,), jnp.float32)
        ov = jnp.ones((LANES,), jnp.float32)

        def fill_zero(t, _):
            work_v[pl.ds(t * LANES, LANES)] = zv
            return 0

        lax.fori_loop(0, ND_T // LANES, fill_zero, 0)
        pltpu.sync_copy(work_v, deg_sh.at[pl.ds(sid * ND_T, ND_T)])

        for t in range(K // LANES):
            ones_v[pl.ds(t * LANES, LANES)] = ov

        plsc.subcore_barrier()

        didxs = [didx, didx2]

        def scat(j2, _):
            for b in range(2):
                j = 2 * j2 + b

                @pl.when(j >= 2)
                def _():
                    pltpu.make_async_copy(ones_v, deg_sh.at[didxs[b]],
                                          dsem.at[b]).wait()

                for t in range(K // LANES):
                    didxs[b][pl.ds(t * LANES, LANES)] = (
                        row1d[pl.ds(j * K + t * LANES, LANES)])
                pltpu.async_copy(ones_v, deg_sh.at[didxs[b]], dsem.at[b],
                                 add=True)
            return 0

        lax.fori_loop(0, EPT // K // 2, scat, 0)
        for b in range(2):
            pltpu.make_async_copy(ones_v, deg_sh.at[didxs[b]],
                                  dsem.at[b]).wait()

        plsc.subcore_barrier()

        pltpu.sync_copy(deg_sh.at[pl.ds(sid * ND_T, ND_T)],
                        deg_hbm.at[pl.ds(sid * ND_T, ND_T)])


@functools.partial(
    pl.kernel,
    out_type=jax.ShapeDtypeStruct((E,), jnp.float32),
    mesh=_mesh,
    scratch_types=[
        pltpu.VMEM((2, N), jnp.float32),
        pltpu.VMEM((N,), jnp.float32),
        pltpu.VMEM((EPW,), jnp.int32),
        pltpu.VMEM((EPW,), jnp.int32),
        pltpu.VMEM((EPW,), jnp.float32),
    ],
    compiler_params=_sc_params,
)
def _coeff_kernel(ab_hbm, nd_hbm, row_hbm, col_hbm, co_hbm,
                  ab_v, nd_v, row_v, col_v, co_v):
    cid = lax.axis_index("c")
    sid = lax.axis_index("s")
    wid = cid * NS + sid
    ebase = wid * EPW

    pltpu.sync_copy(ab_hbm, ab_v)
    pltpu.sync_copy(nd_hbm.at[pl.ds(0, N)], nd_v)
    pltpu.sync_copy(row_hbm.at[pl.ds(ebase, EPW)], row_v)
    pltpu.sync_copy(col_hbm.at[pl.ds(ebase, EPW)], col_v)

    @plsc.parallel_loop(0, EPW // LANES, unroll=4)
    def _(t):
        ridx = row_v[pl.ds(t * LANES, LANES)]
        cidx = col_v[pl.ds(t * LANES, LANES)]
        zero16 = ridx - ridx
        av = plsc.load_gather(ab_v, [zero16, ridx])
        bv = plsc.load_gather(ab_v, [zero16 + 1, cidx])
        nr = plsc.load_gather(nd_v, [ridx])
        ncv = plsc.load_gather(nd_v, [cidx])
        s = av + bv
        s = jnp.minimum(jnp.maximum(s, -10.0), 10.0)
        e2 = jnp.exp(2.0 * s)
        co_v[pl.ds(t * LANES, LANES)] = (e2 - 1.0) / (e2 + 1.0) * nr * ncv
    pltpu.sync_copy(co_v, co_hbm.at[pl.ds(ebase, EPW)])


@functools.partial(
    pl.kernel,
    out_type=jax.ShapeDtypeStruct((NC, N, D), jnp.float32),
    mesh=_mesh,
    scratch_types=[
        [pltpu.VMEM((K,), jnp.int32) for _ in range(NBUF)],
        [pltpu.VMEM((K,), jnp.int32) for _ in range(NBUF)],
        [pltpu.VMEM((K,), jnp.float32) for _ in range(NBUF)],
        pltpu.VMEM((NBUF, K, D), jnp.float32),
        pltpu.VMEM((ZR, D), jnp.float32),
        pltpu.VMEM_SHARED((N, D), jnp.float32),
        pltpu.SemaphoreType.DMA((NBUF,)),
        pltpu.SemaphoreType.DMA((NBUF,)),
        pltpu.SemaphoreType.DMA((NBUF,)),
    ],
    compiler_params=_sc_params,
)
def _mp_kernel(h_hbm, co_hbm, row_hbm, col_hbm, out_hbm,
               rbufs, cbufs, fbufs, hbuf, zbuf, acc, isem, gsem, ssem):
    cid = lax.axis_index("c")
    sid = lax.axis_index("s")
    wid = cid * NS + sid
    ebase = wid * EPW

    zv = jnp.zeros((LANES,), jnp.float32)

    def fill_zero(r, _):
        for q in range(D // LANES):
            zbuf[r, pl.ds(q * LANES, LANES)] = zv
        return 0

    lax.fori_loop(0, ZR, fill_zero, 0)

    @pl.when(sid < 15)
    def _():
        for kz in range(RT_BIG // ZR):
            pltpu.sync_copy(zbuf, acc.at[pl.ds(sid * RT_BIG + kz * ZR, ZR)])

    @pl.when(sid == 15)
    def _():
        for kz in range(RT_SMALL // ZR):
            pltpu.sync_copy(zbuf, acc.at[pl.ds(15 * RT_BIG + kz * ZR, ZR)])

    plsc.subcore_barrier()

    def start_idx(c, p):
        off = ebase + c * K
        pltpu.async_copy(row_hbm.at[pl.ds(off, K)], rbufs[p], isem.at[p])
        pltpu.async_copy(col_hbm.at[pl.ds(off, K)], cbufs[p], isem.at[p])
        pltpu.async_copy(co_hbm.at[pl.ds(off, K)], fbufs[p], isem.at[p])

    def wait_idx(p):
        pltpu.make_async_copy(row_hbm.at[pl.ds(0, K)], rbufs[p],
                              isem.at[p]).wait()
        pltpu.make_async_copy(col_hbm.at[pl.ds(0, K)], cbufs[p],
                              isem.at[p]).wait()
        pltpu.make_async_copy(co_hbm.at[pl.ds(0, K)], fbufs[p],
                              isem.at[p]).wait()

    def start_gather(p):
        pltpu.async_copy(h_hbm.at[rbufs[p]], hbuf.at[p], gsem.at[p])

    def wait_gather(p):
        pltpu.make_async_copy(h_hbm.at[rbufs[p]], hbuf.at[p],
                              gsem.at[p]).wait()

    def start_scatter(p):
        pltpu.async_copy(hbuf.at[p], acc.at[cbufs[p]], ssem.at[p], add=True)

    def wait_scatter(p):
        pltpu.make_async_copy(hbuf.at[p], acc.at[cbufs[p]], ssem.at[p]).wait()

    def compute_chunk(p):
        hb = hbuf.at[p]
        fb = fbufs[p]

        def group(t, _):
            coeff = fb[pl.ds(t * LANES, LANES)]
            for i in range(LANES):
                e = t * LANES + i
                sc = coeff[i]
                for q in range(D // LANES):
                    sl = pl.ds(q * LANES, LANES)
                    hb[e, sl] = hb[e, sl] * sc
            return 0

        lax.fori_loop(0, K // LANES, group, 0)

    start_idx(0, 0)
    start_idx(1, 1)
    wait_idx(0)
    start_gather(0)

    def outer(i4, _):
        for p in range(NBUF):
            c = i4 * NBUF + p

            @pl.when(c + 2 <= NCHUNK - 1)
            def _():
                @pl.when(c >= 2)
                def _():
                    wait_scatter((p + 2) % NBUF)

                start_idx(c + 2, (p + 2) % NBUF)

            wait_idx((p + 1) % NBUF)
            start_gather((p + 1) % NBUF)

            wait_gather(p)
            compute_chunk(p)
            start_scatter(p)
        return 0

    lax.fori_loop(0, (NCHUNK - 1) // NBUF, outer, 0)

    wait_gather(0)
    compute_chunk(0)
    start_scatter(0)
    for p in range(NBUF):
        wait_scatter((1 + p) % NBUF)

    plsc.subcore_barrier()

    @pl.when(sid < 15)
    def _():
        pltpu.sync_copy(acc.at[pl.ds(sid * RT_BIG, RT_BIG)],
                        out_hbm.at[cid].at[pl.ds(sid * RT_BIG, RT_BIG)])

    @pl.when(sid == 15)
    def _():
        pltpu.sync_copy(acc.at[pl.ds(15 * RT_BIG, RT_SMALL)],
                        out_hbm.at[cid].at[pl.ds(15 * RT_BIG, RT_SMALL)])


def _prep0_body(x_ref, w_ref, bias_ref, deg_ref, ab_ref, nd_ref):
    ab = lax.dot_general(w_ref[...], x_ref[...], (((0,), (1,)), ((), ())),
                         preferred_element_type=jnp.float32)
    ab_ref[...] = ab + bias_ref[...][:, :1]
    nd_ref[...] = lax.rsqrt(jnp.maximum(deg_ref[...], 1.0))


def _prep1_body(x_ref, part_ref, w_ref, bias_ref, h_ref, ab_ref):
    h = EPS * x_ref[...] + part_ref[0] + part_ref[1]
    h_ref[...] = h
    ab = lax.dot_general(w_ref[...], h, (((0,), (1,)), ((), ())),
                         preferred_element_type=jnp.float32)
    ab_ref[...] = ab + bias_ref[...][:, :1]


def _final_body(x_ref, part_ref, w_ref, bias_ref, out_ref):
    h = EPS * x_ref[...] + part_ref[0] + part_ref[1]
    out_ref[...] = jnp.dot(h, w_ref[...],
                           preferred_element_type=jnp.float32) + bias_ref[...]


def _prep0(x, wcat, bias2, deg2d):
    return pl.pallas_call(
        _prep0_body,
        out_shape=[
            jax.ShapeDtypeStruct((2, N), jnp.float32),
            jax.ShapeDtypeStruct((N_PAD // 128, 128), jnp.float32),
        ],
    )(x, wcat, bias2, deg2d)


def _prep1(x, part, wcat, bias2):
    return pl.pallas_call(
        _prep1_body,
        out_shape=[
            jax.ShapeDtypeStruct((N, D), jnp.float32),
            jax.ShapeDtypeStruct((2, N), jnp.float32),
        ],
    )(x, part, wcat, bias2)


def _final(x, part, w_head, bias):
    return pl.pallas_call(
        _final_body,
        out_shape=jax.ShapeDtypeStruct((N, OUT), jnp.float32),
    )(x, part, w_head, bias)


def kernel(x, edge_index, W_gate, b_gate, W_head, b_head):
    row = edge_index[0]
    col = edge_index[1]

    deg = _deg_kernel(row)
    deg2d = deg.reshape(N_PAD // 128, 128)

    wg1 = W_gate[:, :D, 0]
    wg2 = W_gate[:, D:, 0]

    part = None
    nd = None
    h = x
    for i in range(L):
        wcat = jnp.stack([wg1[i], wg2[i]], axis=1)
        bias2 = jnp.broadcast_to(
            jnp.concatenate([b_gate[i], jnp.zeros((1,), jnp.float32)])[:, None],
            (2, 128))
        if i == 0:
            ab, nd2d = _prep0(x, wcat, bias2, deg2d)
            nd = nd2d.reshape(N_PAD)
        else:
            h, ab = _prep1(x, part, wcat, bias2)
        co = _coeff_kernel(ab, nd, row, col)
        part = _mp_kernel(h, co, row, col)

    return _final(x, part, W_head, jnp.reshape(b_head, (1, OUT)))
